# baseline (speedup 1.0000x reference)
import functools

import jax
import jax.numpy as jnp
from jax import lax
from jax.experimental import pallas as pl
from jax.experimental.pallas import tpu as pltpu
from jax.experimental.pallas import tpu_sc as plsc

_N = 1024
_NR = 512
_NW = 16
_CPT = 64


def _pack_body(hin_ref, hp_ref):
    sh32 = lax.broadcasted_iota(jnp.int32, (32, 1), 0)
    for w in range(_NW):
        blk = hin_ref[pl.ds(32 * w, 32), :].astype(jnp.int32)
        hp_ref[pl.ds(w, 1), :] = jnp.sum(blk << sh32, axis=0, keepdims=True)


_sc_mesh = plsc.VectorSubcoreMesh(core_axis_name="c", subcore_axis_name="s")


def _lane_perm(x, idx):
    dnums = lax.GatherDimensionNumbers(
        offset_dims=(), collapsed_slice_dims=(0,), start_index_map=(0,))
    return lax.gather(x, idx[:, None], dnums, (1,),
                      mode=lax.GatherScatterMode.PROMISE_IN_BOUNDS)


@functools.partial(
    pl.kernel,
    out_type=jax.ShapeDtypeStruct((_NW, _N), jnp.int32),
    mesh=_sc_mesh,
    scratch_types=[
        pltpu.VMEM((_N,), jnp.int32),
        pltpu.VMEM((_NW,), jnp.int32),
        pltpu.VMEM((_NW,), jnp.int32),
        pltpu.VMEM((_NW,), jnp.int32),
        pltpu.VMEM_SHARED((2, _NW, _NW), jnp.int32),
    ],
)
def _p1_sc(hin, hout, flat, ctmp, rtmp, pivr, shared):
    i32 = jnp.int32
    s = lax.axis_index("s")
    core = lax.axis_index("c")
    iota16 = lax.broadcasted_iota(i32, (_NW,), 0)
    zero16 = iota16 * 0

    pltpu.sync_copy(hin.at[s], flat)

    def publish(col, slot):
        jl = col & (_CPT - 1)
        ctmp[...] = flat[pl.ds(_NW * jl, _NW)]
        pltpu.sync_copy(ctmp, shared.at[slot, s])

    pivr[...] = zero16 + _NR
    publish(i32(_N - 1), i32(0))
    plsc.subcore_barrier()

    iota_x = [iota16 ^ k for k in (1, 2, 4, 8)]

    def allmax(v):
        for ix in iota_x:
            v = jnp.maximum(v, _lane_perm(v, ix))
        return v

    def shrl(v, k):
        return lax.shift_right_logical(v, zero16 + k)

    def body(i, _):
        piv = pivr[...]
        col = _N - 1 - i
        pltpu.sync_copy(shared.at[lax.rem(i, 2), col >> 6], rtmp)
        c = rtmp[...]
        sh = piv - 32 * iota16
        wm = jnp.where(sh >= 32, i32(-1),
                       (i32(1) << jnp.maximum(jnp.minimum(sh, 31), 0)) - 1)
        masked = c & wm
        x = masked
        x = x | shrl(x, 1)
        x = x | shrl(x, 2)
        x = x | shrl(x, 4)
        x = x | shrl(x, 8)
        x = x | shrl(x, 16)
        top = x ^ shrl(x, 1)
        posw = (jnp.where((top & i32(-1431655766)) != 0, i32(1), i32(0))
                + jnp.where((top & i32(-858993460)) != 0, i32(2), i32(0))
                + jnp.where((top & i32(-252645136)) != 0, i32(4), i32(0))
                + jnp.where((top & i32(-16711936)) != 0, i32(8), i32(0))
                + jnp.where((top & i32(-65536)) != 0, i32(16), i32(0)))
        rowcand = jnp.where(masked != 0, 32 * iota16 + posw, -1)
        r1v = allmax(rowcand)
        fmask = jnp.where((r1v >= 0) & (piv > 0), i32(-1), i32(0))
        r1c = jnp.maximum(r1v, 0)
        w1v, b1v = r1c >> 5, r1c & 31
        r2v = jnp.maximum(piv - 1, 0)
        w2v, b2v = r2v >> 5, r2v & 31
        clr = (jnp.where(iota16 == w1v, i32(1) << b1v, 0)
               | jnp.where(iota16 == w2v, i32(1) << b2v, 0))
        pc = (c & ~clr) & fmask
        clrf = clr & fmask
        for jl in range(_CPT):
            cv = flat[pl.ds(_NW * jl, _NW)]
            g1 = _lane_perm(cv, w1v)
            g2 = _lane_perm(cv, w2v)
            bit1 = (g1 >> b1v) & 1
            dd = bit1 ^ ((g2 >> b2v) & 1)
            cv = cv ^ ((0 - dd) & clrf) ^ (pc & (0 - bit1))
            flat[pl.ds(_NW * jl, _NW)] = cv
        pivr[...] = jnp.where(fmask != 0, piv - 1, piv)
        publish(col - 1, lax.rem(i + 1, 2))
        plsc.subcore_barrier()
        return i32(0)

    lax.fori_loop(0, _N, body, i32(0))

    @pl.when(core == 0)
    def _():
        pltpu.sync_copy(flat, hout.at[s])


def _asm_body(hf_ref, g_ref, src_ref, u_ref, m_ref):
    i32 = jnp.int32
    f32 = jnp.float32
    sh32 = lax.broadcasted_iota(i32, (32, 1), 0)
    iota_w = lax.broadcasted_iota(i32, (_NW, 1), 0)
    lane_n = lax.broadcasted_iota(i32, (1, _N), 1)
    Hf = hf_ref[:, :]

    wnz = Hf != 0
    single = (Hf & (Hf - 1)) == 0
    all_single = jnp.all(single, axis=0, keepdims=True)
    nzcnt = jnp.sum(wnz.astype(i32), axis=0, keepdims=True)
    posw = (((Hf & i32(-1431655766)) != 0).astype(i32)
            + (((Hf & i32(-858993460)) != 0).astype(i32) << 1)
            + (((Hf & i32(-252645136)) != 0).astype(i32) << 2)
            + (((Hf & i32(-16711936)) != 0).astype(i32) << 3)
            + (((Hf & i32(-65536)) != 0).astype(i32) << 4))
    rowpos = jnp.sum(jnp.where(wnz, iota_w * 32 + posw, 0),
                     axis=0, keepdims=True)
    colid0 = jnp.where(all_single & (nzcnt == 1), rowpos, -1)

    big = i32(1) << 20
    i_col = lax.broadcasted_iota(i32, (_NR, 1), 0)
    fm = jnp.min(jnp.where(colid0 == i_col, lane_n, big),
                 axis=1, keepdims=True)
    ok = (fm == _N - _NR + i_col) | (fm >= big)
    i0 = jnp.min(jnp.where(ok, _NR, i_col))

    def p2_cond(carry):
        return carry[0] < _NR

    def p2_body(carry):
        i, colid, csv = carry
        m = colid == i
        cond = jnp.any(m)
        jstar = jnp.min(jnp.where(m, lane_n, big))
        jstar = jnp.where(cond, jstar, 0)
        cei = _N - _NR + i
        mA = lane_n == jstar
        mB = lane_n == cei
        cidB = jnp.sum(jnp.where(mB, colid, 0))
        cid_sw = jnp.where(mA, cidB, jnp.where(mB, i, colid))
        colid = jnp.where(cond, cid_sw, colid)
        sA = jnp.sum(jnp.where(mA, csv, 0))
        sB = jnp.sum(jnp.where(mB, csv, 0))
        cs_sw = jnp.where(mA, sB, jnp.where(mB, sA, csv))
        return i + 1, colid, jnp.where(cond, cs_sw, csv)

    _, _, csv = lax.while_loop(
        p2_cond, p2_body,
        (i0, colid0, lax.broadcasted_iota(i32, (1, _N), 1)))

    for w in range(_NW):
        g_ref[pl.ds(32 * w, 32), :] = (
            (Hf[w:w + 1, :] >> sh32) & 1).astype(f32)
    perm = (lax.broadcasted_iota(i32, (_N, _NR), 0)
            == csv[:, 0:_NR]).astype(f32)
    u_ref[:, :] = lax.dot_general(g_ref[:, :], perm,
                                  (((1,), (0,)), ((), ())),
                                  preferred_element_type=f32)
    csf = csv.astype(f32)
    eye = (lax.broadcasted_iota(i32, (_NR, _NR), 0)
           == lax.broadcasted_iota(i32, (_NR, _NR), 1)).astype(f32)
    cs_top = lax.dot_general(eye, csf[:, 0:_NR],
                             (((1,), (1,)), ((), ())),
                             preferred_element_type=f32)
    cs_bot = lax.dot_general(eye, csf[:, _NR:_N],
                             (((1,), (1,)), ((), ())),
                             preferred_element_type=f32)
    lane_ni = lax.broadcasted_iota(i32, (_NR, _N), 1)
    m_ref[:, :] = (cs_bot.astype(i32) == lane_ni).astype(f32)
    g_ref[:, :] = ((cs_top.astype(i32) == lane_ni).astype(f32)
                   + lax.dot_general(u_ref[:, :], m_ref[:, :],
                                     (((0,), (0,)), ((), ())),
                                     preferred_element_type=f32))
    src_ref[0:1, :] = csv[0:1, 0:_NR]


def kernel(H_input):
    planes = pl.pallas_call(
        _pack_body,
        out_shape=jax.ShapeDtypeStruct((_NW, _N), jnp.int32),
    )(H_input)
    packed = planes.reshape(_NW, _NW, _CPT).transpose(1, 2, 0).reshape(
        _NW, _N)
    hf3 = _p1_sc(packed)
    hf = hf3.reshape(_NW, _CPT, _NW).transpose(2, 0, 1).reshape(_NW, _N)
    g, src = pl.pallas_call(
        _asm_body,
        out_shape=[
            jax.ShapeDtypeStruct((_NR, _N), jnp.float32),
            jax.ShapeDtypeStruct((1, _NR), jnp.int32),
        ],
        scratch_shapes=[
            pltpu.VMEM((_NR, _NR), jnp.float32),
            pltpu.VMEM((_NR, _N), jnp.float32),
        ],
    )(hf)
    return g, src.reshape(_NR)


# speedup vs baseline: 1.0585x; 1.0585x over previous
import functools

import jax
import jax.numpy as jnp
from jax import lax
from jax.experimental import pallas as pl
from jax.experimental.pallas import tpu as pltpu
from jax.experimental.pallas import tpu_sc as plsc

_N = 1024
_NR = 512
_NW = 16
_CPT = 64


def _pack_body(hin_ref, hp_ref):
    sh32 = lax.broadcasted_iota(jnp.int32, (32, 1), 0)
    for w in range(_NW):
        blk = hin_ref[pl.ds(32 * w, 32), :].astype(jnp.int32)
        hp_ref[pl.ds(w, 1), :] = jnp.sum(blk << sh32, axis=0, keepdims=True)


_sc_mesh = plsc.VectorSubcoreMesh(core_axis_name="c", subcore_axis_name="s")


def _lane_perm(x, idx):
    dnums = lax.GatherDimensionNumbers(
        offset_dims=(), collapsed_slice_dims=(0,), start_index_map=(0,))
    return lax.gather(x, idx[:, None], dnums, (1,),
                      mode=lax.GatherScatterMode.PROMISE_IN_BOUNDS)


@functools.partial(
    pl.kernel,
    out_type=jax.ShapeDtypeStruct((_NW, _N), jnp.int32),
    mesh=_sc_mesh,
    scratch_types=[
        pltpu.VMEM((_N,), jnp.int32),
        pltpu.VMEM((_NW,), jnp.int32),
        pltpu.VMEM((_NW,), jnp.int32),
        pltpu.VMEM((_NW,), jnp.int32),
        pltpu.VMEM_SHARED((2, _NW, _NW), jnp.int32),
        pltpu.SemaphoreType.DMA,
    ],
)
def _p1_sc(hin, hout, flat, ctmp, rtmp, pivr, shared, dsem):
    i32 = jnp.int32
    s = lax.axis_index("s")
    core = lax.axis_index("c")
    iota16 = lax.broadcasted_iota(i32, (_NW,), 0)
    zero16 = iota16 * 0

    pltpu.sync_copy(hin.at[s], flat)

    def publish(col, slot):
        jl = col & (_CPT - 1)
        ctmp[...] = flat[pl.ds(_NW * jl, _NW)]
        pltpu.sync_copy(ctmp, shared.at[slot, s])

    pivr[...] = zero16 + _NR
    publish(i32(_N - 1), i32(0))
    plsc.subcore_barrier()

    iota_x = [iota16 ^ k for k in (1, 2, 4, 8)]

    def allmax(v):
        for ix in iota_x:
            v = jnp.maximum(v, _lane_perm(v, ix))
        return v

    def shrl(v, k):
        return lax.shift_right_logical(v, zero16 + k)

    def body(i, _):
        piv = pivr[...]
        col = _N - 1 - i
        pltpu.sync_copy(shared.at[lax.rem(i, 2), col >> 6], rtmp)
        c = rtmp[...]
        sh = piv - 32 * iota16
        wm = jnp.where(sh >= 32, i32(-1),
                       (i32(1) << jnp.maximum(jnp.minimum(sh, 31), 0)) - 1)
        masked = c & wm
        x = masked
        x = x | shrl(x, 1)
        x = x | shrl(x, 2)
        x = x | shrl(x, 4)
        x = x | shrl(x, 8)
        x = x | shrl(x, 16)
        top = x ^ shrl(x, 1)
        posw = (jnp.where((top & i32(-1431655766)) != 0, i32(1), i32(0))
                + jnp.where((top & i32(-858993460)) != 0, i32(2), i32(0))
                + jnp.where((top & i32(-252645136)) != 0, i32(4), i32(0))
                + jnp.where((top & i32(-16711936)) != 0, i32(8), i32(0))
                + jnp.where((top & i32(-65536)) != 0, i32(16), i32(0)))
        rowcand = jnp.where(masked != 0, 32 * iota16 + posw, -1)
        r1v = allmax(rowcand)
        fmask = jnp.where((r1v >= 0) & (piv > 0), i32(-1), i32(0))
        r1c = jnp.maximum(r1v, 0)
        w1v, b1v = r1c >> 5, r1c & 31
        r2v = jnp.maximum(piv - 1, 0)
        w2v, b2v = r2v >> 5, r2v & 31
        clr = (jnp.where(iota16 == w1v, i32(1) << b1v, 0)
               | jnp.where(iota16 == w2v, i32(1) << b2v, 0))
        pc = (c & ~clr) & fmask
        clrf = clr & fmask

        def colstep(cv):
            g1 = _lane_perm(cv, w1v)
            g2 = _lane_perm(cv, w2v)
            bit1 = (g1 >> b1v) & 1
            dd = bit1 ^ ((g2 >> b2v) & 1)
            return ((0 - dd) & clrf) ^ (pc & (0 - bit1))

        jl2 = (col - 1) & (_CPT - 1)
        o2 = _NW * jl2
        cv2 = flat[pl.ds(o2, _NW)]
        cv2 = cv2 ^ colstep(cv2)
        flat[pl.ds(o2, _NW)] = cv2
        ctmp[...] = cv2
        cp = pltpu.async_copy(ctmp, shared.at[lax.rem(i + 1, 2), s], dsem)
        for jl in range(_CPT):
            skip = jnp.where(jl2 == jl, i32(0), i32(-1))
            cv = flat[pl.ds(_NW * jl, _NW)]
            cv = cv ^ (colstep(cv) & skip)
            flat[pl.ds(_NW * jl, _NW)] = cv
        cp.wait()
        pivr[...] = jnp.where(fmask != 0, piv - 1, piv)
        plsc.subcore_barrier()
        return i32(0)

    lax.fori_loop(0, _N, body, i32(0))

    @pl.when(core == 0)
    def _():
        pltpu.sync_copy(flat, hout.at[s])


def _asm_body(hf_ref, g_ref, src_ref, u_ref, m_ref):
    i32 = jnp.int32
    f32 = jnp.float32
    sh32 = lax.broadcasted_iota(i32, (32, 1), 0)
    iota_w = lax.broadcasted_iota(i32, (_NW, 1), 0)
    lane_n = lax.broadcasted_iota(i32, (1, _N), 1)
    Hf = hf_ref[:, :]

    wnz = Hf != 0
    single = (Hf & (Hf - 1)) == 0
    all_single = jnp.all(single, axis=0, keepdims=True)
    nzcnt = jnp.sum(wnz.astype(i32), axis=0, keepdims=True)
    posw = (((Hf & i32(-1431655766)) != 0).astype(i32)
            + (((Hf & i32(-858993460)) != 0).astype(i32) << 1)
            + (((Hf & i32(-252645136)) != 0).astype(i32) << 2)
            + (((Hf & i32(-16711936)) != 0).astype(i32) << 3)
            + (((Hf & i32(-65536)) != 0).astype(i32) << 4))
    rowpos = jnp.sum(jnp.where(wnz, iota_w * 32 + posw, 0),
                     axis=0, keepdims=True)
    colid0 = jnp.where(all_single & (nzcnt == 1), rowpos, -1)

    big = i32(1) << 20
    i_col = lax.broadcasted_iota(i32, (_NR, 1), 0)
    fm = jnp.min(jnp.where(colid0 == i_col, lane_n, big),
                 axis=1, keepdims=True)
    ok = (fm == _N - _NR + i_col) | (fm >= big)
    i0 = jnp.min(jnp.where(ok, _NR, i_col))

    def p2_cond(carry):
        return carry[0] < _NR

    def p2_body(carry):
        i, colid, csv = carry
        m = colid == i
        cond = jnp.any(m)
        jstar = jnp.min(jnp.where(m, lane_n, big))
        jstar = jnp.where(cond, jstar, 0)
        cei = _N - _NR + i
        mA = lane_n == jstar
        mB = lane_n == cei
        cidB = jnp.sum(jnp.where(mB, colid, 0))
        cid_sw = jnp.where(mA, cidB, jnp.where(mB, i, colid))
        colid = jnp.where(cond, cid_sw, colid)
        sA = jnp.sum(jnp.where(mA, csv, 0))
        sB = jnp.sum(jnp.where(mB, csv, 0))
        cs_sw = jnp.where(mA, sB, jnp.where(mB, sA, csv))
        return i + 1, colid, jnp.where(cond, cs_sw, csv)

    _, _, csv = lax.while_loop(
        p2_cond, p2_body,
        (i0, colid0, lax.broadcasted_iota(i32, (1, _N), 1)))

    for w in range(_NW):
        g_ref[pl.ds(32 * w, 32), :] = (
            (Hf[w:w + 1, :] >> sh32) & 1).astype(f32)
    perm = (lax.broadcasted_iota(i32, (_N, _NR), 0)
            == csv[:, 0:_NR]).astype(f32)
    u_ref[:, :] = lax.dot_general(g_ref[:, :], perm,
                                  (((1,), (0,)), ((), ())),
                                  preferred_element_type=f32)
    csf = csv.astype(f32)
    eye = (lax.broadcasted_iota(i32, (_NR, _NR), 0)
           == lax.broadcasted_iota(i32, (_NR, _NR), 1)).astype(f32)
    cs_top = lax.dot_general(eye, csf[:, 0:_NR],
                             (((1,), (1,)), ((), ())),
                             preferred_element_type=f32)
    cs_bot = lax.dot_general(eye, csf[:, _NR:_N],
                             (((1,), (1,)), ((), ())),
                             preferred_element_type=f32)
    lane_ni = lax.broadcasted_iota(i32, (_NR, _N), 1)
    m_ref[:, :] = (cs_bot.astype(i32) == lane_ni).astype(f32)
    g_ref[:, :] = ((cs_top.astype(i32) == lane_ni).astype(f32)
                   + lax.dot_general(u_ref[:, :], m_ref[:, :],
                                     (((0,), (0,)), ((), ())),
                                     preferred_element_type=f32))
    src_ref[0:1, :] = csv[0:1, 0:_NR]


def kernel(H_input):
    planes = pl.pallas_call(
        _pack_body,
        out_shape=jax.ShapeDtypeStruct((_NW, _N), jnp.int32),
    )(H_input)
    packed = planes.reshape(_NW, _NW, _CPT).transpose(1, 2, 0).reshape(
        _NW, _N)
    hf3 = _p1_sc(packed)
    hf = hf3.reshape(_NW, _CPT, _NW).transpose(2, 0, 1).reshape(_NW, _N)
    g, src = pl.pallas_call(
        _asm_body,
        out_shape=[
            jax.ShapeDtypeStruct((_NR, _N), jnp.float32),
            jax.ShapeDtypeStruct((1, _NR), jnp.int32),
        ],
        scratch_shapes=[
            pltpu.VMEM((_NR, _NR), jnp.float32),
            pltpu.VMEM((_NR, _N), jnp.float32),
        ],
    )(hf)
    return g, src.reshape(_NR)
